# baseline (speedup 1.0000x reference)
import functools

import jax
import jax.numpy as jnp
from jax.experimental import pallas as pl
from jax.experimental.pallas import tpu as pltpu

B, S, H, E, K, I = 1, 2048, 768, 8, 2, 3072
BALANCE_COEF = 0.01
N = B * S * K
TM = 256
NB = N // TM
G = NB + E - 1
LANES = 128


def _router_body(x_ref, wg_ref, idx_ref, w_ref, counts_ref, loss_ref):
    x = x_ref[...]
    wg = wg_ref[...]
    logits = jax.lax.dot_general(
        x, wg, (((1,), (0,)), ((), ())), preferred_element_type=jnp.float32)
    lane = jax.lax.broadcasted_iota(jnp.int32, (S, LANES), 1)
    valid = lane < E
    lg = jnp.where(valid, logits, -1e30)
    m = jnp.max(lg, axis=1, keepdims=True)
    p = jnp.where(valid, jnp.exp(lg - m), 0.0)
    probs = p / jnp.sum(p, axis=1, keepdims=True)
    v1 = jnp.max(probs, axis=1, keepdims=True)
    i1 = jnp.min(jnp.where(probs == v1, lane, LANES), axis=1, keepdims=True)
    probs_m = jnp.where(lane == i1, -1.0, probs)
    v2 = jnp.max(probs_m, axis=1, keepdims=True)
    i2 = jnp.min(jnp.where(probs_m == v2, lane, LANES), axis=1, keepdims=True)
    denom = v1 + v2
    idx_ref[...] = jnp.where(lane == 0, i1,
                             jnp.where(lane == 1, i2, 0)).astype(jnp.int32)
    w_ref[...] = jnp.where(lane == 0, v1 / denom,
                           jnp.where(lane == 1, v2 / denom, 0.0))
    onehot = ((lane == i1) | (lane == i2)).astype(jnp.float32)
    counts = jnp.sum(onehot, axis=0, keepdims=True)
    counts_ref[...] = counts.astype(jnp.int32)
    pmean = jnp.mean(probs, axis=0, keepdims=True)
    f = counts / jnp.float32(S)
    loss = BALANCE_COEF * E * jnp.sum(f * pmean)
    lane0 = jax.lax.broadcasted_iota(jnp.int32, (1, LANES), 1)
    loss_ref[...] = jnp.where(lane0 == 0, loss, 0.0)


def _router(x, wg_padded):
    return pl.pallas_call(
        _router_body,
        out_shape=(
            jax.ShapeDtypeStruct((S, LANES), jnp.int32),
            jax.ShapeDtypeStruct((S, LANES), jnp.float32),
            jax.ShapeDtypeStruct((1, LANES), jnp.int32),
            jax.ShapeDtypeStruct((1, LANES), jnp.float32),
        ),
    )(x, wg_padded)


def _ffn_body(cuts_ref, blk_ref, exp_ref, isf_ref, isl_ref,
              x_ref, w1_ref, b1_ref, w2_ref, b2_ref, ws_ref, out_ref):
    g = pl.program_id(0)

    @pl.when(isf_ref[g] == 1)
    def _():
        out_ref[...] = jnp.zeros_like(out_ref)

    x = x_ref[...].astype(jnp.bfloat16)
    h = jnp.dot(x, w1_ref[0].astype(jnp.bfloat16),
                preferred_element_type=jnp.float32) + b1_ref[0]
    h = jax.nn.gelu(h).astype(jnp.bfloat16)
    y = jnp.dot(h, w2_ref[0].astype(jnp.bfloat16),
                preferred_element_type=jnp.float32) + b2_ref[0]
    row = blk_ref[g] * TM + jax.lax.broadcasted_iota(jnp.int32, (TM, 1), 0)
    mask = (row >= cuts_ref[g]) & (row < cuts_ref[g + 1])
    mw = jnp.where(mask, ws_ref[...], 0.0)
    out_ref[...] += mw * y


def _grouped_ffn(cuts, blk_ids, exp_ids, isf, isl, x_sorted, W1, b1r, W2, b2r, ws2d):
    grid_spec = pltpu.PrefetchScalarGridSpec(
        num_scalar_prefetch=5,
        grid=(G,),
        in_specs=[
            pl.BlockSpec((TM, H), lambda g, c, b, e, f, l: (b[g], 0)),
            pl.BlockSpec((1, H, I), lambda g, c, b, e, f, l: (e[g], 0, 0)),
            pl.BlockSpec((1, 1, I), lambda g, c, b, e, f, l: (e[g], 0, 0)),
            pl.BlockSpec((1, I, H), lambda g, c, b, e, f, l: (e[g], 0, 0)),
            pl.BlockSpec((1, 1, H), lambda g, c, b, e, f, l: (e[g], 0, 0)),
            pl.BlockSpec((TM, 1), lambda g, c, b, e, f, l: (b[g], 0)),
        ],
        out_specs=pl.BlockSpec((TM, H), lambda g, c, b, e, f, l: (b[g], 0)),
    )
    return pl.pallas_call(
        _ffn_body,
        grid_spec=grid_spec,
        out_shape=jax.ShapeDtypeStruct((N, H), jnp.float32),
    )(cuts, blk_ids, exp_ids, isf, isl, x_sorted, W1, b1r, W2, b2r, ws2d)


def kernel(hidden_states, Wg, W1, b1, W2, b2):
    x = hidden_states.reshape(S, H)
    wg_padded = jnp.pad(Wg, ((0, 0), (0, LANES - E)))

    idx_out, w_out, counts_out, loss_out = _router(x, wg_padded)
    balance_loss = loss_out[0, 0]
    counts = counts_out[0, :E]
    experts_flat = idx_out[:, :K].reshape(-1)
    weights_flat = w_out[:, :K].reshape(-1)

    offs = jnp.concatenate([jnp.zeros((1,), jnp.int32),
                            jnp.cumsum(counts, dtype=jnp.int32)])
    sort_idx = jnp.argsort(experts_flat).astype(jnp.int32)
    x_sorted = jnp.take(x, sort_idx // K, axis=0)
    ws2d = jnp.take(weights_flat, sort_idx)[:, None]

    blk_bounds = jnp.arange(NB, dtype=jnp.int32) * TM
    cuts = jnp.sort(jnp.concatenate([blk_bounds, offs[1:E]]))
    cuts_full = jnp.concatenate([cuts, jnp.full((1,), N, jnp.int32)])
    blk_ids = jnp.clip(cuts // TM, 0, NB - 1).astype(jnp.int32)
    exp_ids = jnp.clip(jnp.searchsorted(offs, cuts, side="right") - 1,
                       0, E - 1).astype(jnp.int32)
    prev = jnp.concatenate([jnp.full((1,), -1, jnp.int32), blk_ids[:-1]])
    nxt = jnp.concatenate([blk_ids[1:], jnp.full((1,), -1, jnp.int32)])
    isf = (blk_ids != prev).astype(jnp.int32)
    isl = (blk_ids != nxt).astype(jnp.int32)

    b1r = b1[:, None, :]
    b2r = b2[:, None, :]
    y_sorted = _grouped_ffn(cuts_full, blk_ids, exp_ids, isf, isl,
                            x_sorted, W1, b1r, W2, b2r, ws2d)

    inv = jnp.argsort(sort_idx).astype(jnp.int32)
    y_pairs = jnp.take(y_sorted, inv, axis=0).reshape(S, K, H)
    out = (x + y_pairs.sum(axis=1)).reshape(B, S, H)
    return out, balance_loss


# speedup vs baseline: 1.0288x; 1.0288x over previous
import functools

import jax
import jax.numpy as jnp
from jax.experimental import pallas as pl
from jax.experimental.pallas import tpu as pltpu

B, S, H, E, K, I = 1, 2048, 768, 8, 2, 3072
BALANCE_COEF = 0.01
N = B * S * K
TM = 512
NB = N // TM
G = NB + E - 1
LANES = 128


def _router_body(x_ref, wg_ref, idx_ref, w_ref, counts_ref, loss_ref):
    x = x_ref[...]
    wg = wg_ref[...]
    logits = jax.lax.dot_general(
        x, wg, (((1,), (0,)), ((), ())), preferred_element_type=jnp.float32)
    lane = jax.lax.broadcasted_iota(jnp.int32, (S, LANES), 1)
    valid = lane < E
    lg = jnp.where(valid, logits, -1e30)
    m = jnp.max(lg, axis=1, keepdims=True)
    p = jnp.where(valid, jnp.exp(lg - m), 0.0)
    probs = p / jnp.sum(p, axis=1, keepdims=True)
    v1 = jnp.max(probs, axis=1, keepdims=True)
    i1 = jnp.min(jnp.where(probs == v1, lane, LANES), axis=1, keepdims=True)
    probs_m = jnp.where(lane == i1, -1.0, probs)
    v2 = jnp.max(probs_m, axis=1, keepdims=True)
    i2 = jnp.min(jnp.where(probs_m == v2, lane, LANES), axis=1, keepdims=True)
    denom = v1 + v2
    idx_ref[...] = jnp.where(lane == 0, i1,
                             jnp.where(lane == 1, i2, 0)).astype(jnp.int32)
    w_ref[...] = jnp.where(lane == 0, v1 / denom,
                           jnp.where(lane == 1, v2 / denom, 0.0))
    onehot = ((lane == i1) | (lane == i2)).astype(jnp.float32)
    counts = jnp.sum(onehot, axis=0, keepdims=True)
    counts_ref[...] = counts.astype(jnp.int32)
    pmean = jnp.mean(probs, axis=0, keepdims=True)
    f = counts / jnp.float32(S)
    loss = BALANCE_COEF * E * jnp.sum(f * pmean)
    lane0 = jax.lax.broadcasted_iota(jnp.int32, (1, LANES), 1)
    loss_ref[...] = jnp.where(lane0 == 0, loss, 0.0)


def _router(x, wg_padded):
    return pl.pallas_call(
        _router_body,
        out_shape=(
            jax.ShapeDtypeStruct((S, LANES), jnp.int32),
            jax.ShapeDtypeStruct((S, LANES), jnp.float32),
            jax.ShapeDtypeStruct((1, LANES), jnp.int32),
            jax.ShapeDtypeStruct((1, LANES), jnp.float32),
        ),
    )(x, wg_padded)


def _ffn_body(cuts_ref, blk_ref, exp_ref, isf_ref, isl_ref,
              x_ref, w1_ref, b1_ref, w2_ref, b2_ref, ws_ref, out_ref):
    g = pl.program_id(0)

    @pl.when(isf_ref[g] == 1)
    def _():
        out_ref[...] = jnp.zeros_like(out_ref)

    x = x_ref[...].astype(jnp.bfloat16)
    h = jnp.dot(x, w1_ref[0].astype(jnp.bfloat16),
                preferred_element_type=jnp.float32) + b1_ref[0]
    h = jax.nn.gelu(h).astype(jnp.bfloat16)
    y = jnp.dot(h, w2_ref[0].astype(jnp.bfloat16),
                preferred_element_type=jnp.float32) + b2_ref[0]
    row = blk_ref[g] * TM + jax.lax.broadcasted_iota(jnp.int32, (TM, 1), 0)
    mask = (row >= cuts_ref[g]) & (row < cuts_ref[g + 1])
    mw = jnp.where(mask, ws_ref[...], 0.0)
    out_ref[...] += mw * y


def _grouped_ffn(cuts, blk_ids, exp_ids, isf, isl, x_sorted, W1, b1r, W2, b2r, ws2d):
    grid_spec = pltpu.PrefetchScalarGridSpec(
        num_scalar_prefetch=5,
        grid=(G,),
        in_specs=[
            pl.BlockSpec((TM, H), lambda g, c, b, e, f, l: (b[g], 0)),
            pl.BlockSpec((1, H, I), lambda g, c, b, e, f, l: (e[g], 0, 0)),
            pl.BlockSpec((1, 1, I), lambda g, c, b, e, f, l: (e[g], 0, 0)),
            pl.BlockSpec((1, I, H), lambda g, c, b, e, f, l: (e[g], 0, 0)),
            pl.BlockSpec((1, 1, H), lambda g, c, b, e, f, l: (e[g], 0, 0)),
            pl.BlockSpec((TM, 1), lambda g, c, b, e, f, l: (b[g], 0)),
        ],
        out_specs=pl.BlockSpec((TM, H), lambda g, c, b, e, f, l: (b[g], 0)),
    )
    return pl.pallas_call(
        _ffn_body,
        grid_spec=grid_spec,
        out_shape=jax.ShapeDtypeStruct((N, H), jnp.float32),
    )(cuts, blk_ids, exp_ids, isf, isl, x_sorted, W1, b1r, W2, b2r, ws2d)


def kernel(hidden_states, Wg, W1, b1, W2, b2):
    x = hidden_states.reshape(S, H)
    wg_padded = jnp.pad(Wg, ((0, 0), (0, LANES - E)))

    idx_out, w_out, counts_out, loss_out = _router(x, wg_padded)
    balance_loss = loss_out[0, 0]
    counts = counts_out[0, :E]
    experts_flat = idx_out[:, :K].reshape(-1)
    weights_flat = w_out[:, :K].reshape(-1)

    offs = jnp.concatenate([jnp.zeros((1,), jnp.int32),
                            jnp.cumsum(counts, dtype=jnp.int32)])
    sort_idx = jnp.argsort(experts_flat).astype(jnp.int32)
    x_sorted = jnp.take(x, sort_idx // K, axis=0)
    ws2d = jnp.take(weights_flat, sort_idx)[:, None]

    blk_bounds = jnp.arange(NB, dtype=jnp.int32) * TM
    cuts = jnp.sort(jnp.concatenate([blk_bounds, offs[1:E]]))
    cuts_full = jnp.concatenate([cuts, jnp.full((1,), N, jnp.int32)])
    blk_ids = jnp.clip(cuts // TM, 0, NB - 1).astype(jnp.int32)
    exp_ids = jnp.clip(jnp.searchsorted(offs, cuts, side="right") - 1,
                       0, E - 1).astype(jnp.int32)
    prev = jnp.concatenate([jnp.full((1,), -1, jnp.int32), blk_ids[:-1]])
    nxt = jnp.concatenate([blk_ids[1:], jnp.full((1,), -1, jnp.int32)])
    isf = (blk_ids != prev).astype(jnp.int32)
    isl = (blk_ids != nxt).astype(jnp.int32)

    b1r = b1[:, None, :]
    b2r = b2[:, None, :]
    y_sorted = _grouped_ffn(cuts_full, blk_ids, exp_ids, isf, isl,
                            x_sorted, W1, b1r, W2, b2r, ws2d)

    inv = jnp.argsort(sort_idx).astype(jnp.int32)
    y_pairs = jnp.take(y_sorted, inv, axis=0).reshape(S, K, H)
    out = (x + y_pairs.sum(axis=1)).reshape(B, S, H)
    return out, balance_loss
